# baseline (speedup 1.0000x reference)
import functools

import jax
import jax.numpy as jnp
from jax import lax
from jax.experimental import pallas as pl
from jax.experimental.pallas import tpu as pltpu
from jax.experimental.pallas import tpu_sc as plsc

N_IMG = 16
L = 16384
N = N_IMG * L
K = max(1, int(L * 0.02))
N_POS = N_IMG * K
C_CLS = 17
C_BOX = 5
R128 = N // 128
BR = 64
NB = R128 // BR

SC_CORES = 2
SC_SUB = 16
IMG_PER_CORE = N_IMG // SC_CORES
HALF = L // 2
NV_FULL = L // 16
NV_HALF = HALF // 16
NBINS = 1024


def _sigmoid(x):
    return jax.nn.sigmoid(x)


def _stream_kernel(t_cls_ref, s_cls_ref, t_box_ref, s_box_ref,
                   t_cent_ref, s_cent_ref,
                   max_ref, maxbits_ref, diff_ref, c_ref, d_ref, neg_ref,
                   w17_ref, w5_ref):
    i = pl.program_id(0)

    @pl.when(i == 0)
    def _init():
        r17 = lax.broadcasted_iota(jnp.int32, (128 * C_CLS, 128), 0)
        c17 = lax.broadcasted_iota(jnp.int32, (128 * C_CLS, 128), 1)
        w17_ref[...] = ((r17 >= C_CLS * c17) & (r17 < C_CLS * c17 + C_CLS)
                        ).astype(jnp.float32)
        r5 = lax.broadcasted_iota(jnp.int32, (128 * C_BOX, 128), 0)
        c5 = lax.broadcasted_iota(jnp.int32, (128 * C_BOX, 128), 1)
        w5_ref[...] = ((r5 >= C_BOX * c5) & (r5 < C_BOX * c5 + C_BOX)
                       ).astype(jnp.float32)
        neg_ref[0, 0] = 0.0

    t = t_cls_ref[...]
    s = s_cls_ref[...]
    s_sig = _sigmoid(s)
    t_sig = _sigmoid(t)
    ls = jnp.clip(jnp.log(s_sig), -100.0, None)
    l1s = jnp.clip(jnp.log(1.0 - s_sig), -100.0, None)
    neg = -l1s * (s_sig * s_sig)
    dts = t_sig - s_sig
    pos = -(t_sig * ls + (1.0 - t_sig) * l1s) * (dts * dts)
    neg_ref[0, 0] += jnp.sum(neg)
    diff_ref[...] = jnp.dot(pos - neg, w17_ref[...],
                            preferred_element_type=jnp.float32,
                            precision=lax.Precision.HIGHEST)

    lane = lax.broadcasted_iota(jnp.int32, (BR, 128 * C_CLS), 1)
    posn = lane % C_CLS
    m = t
    for d in (1, 2, 4, 8, 16):
        shifted = pltpu.roll(m, 128 * C_CLS - d, 1)
        m = jnp.maximum(m, jnp.where(posn <= C_CLS - 1 - d, shifted,
                                     -jnp.inf))
    head = jnp.where(posn == 0, m, 0.0)
    maxv = _sigmoid(jnp.dot(head, w17_ref[...],
                            preferred_element_type=jnp.float32,
                            precision=lax.Precision.HIGHEST))
    max_ref[...] = maxv
    maxbits_ref[...] = lax.bitcast_convert_type(maxv, jnp.int32)

    dbox = jnp.abs(s_box_ref[...] - t_box_ref[...])
    sl1 = jnp.where(dbox < 1.0, 0.5 * dbox * dbox, dbox - 0.5)
    tcs = _sigmoid(t_cent_ref[...])
    c_ref[...] = jnp.dot(sl1, w5_ref[...],
                         preferred_element_type=jnp.float32,
                            precision=lax.Precision.HIGHEST) * tcs

    scs = _sigmoid(s_cent_ref[...])
    lp = jnp.clip(jnp.log(scs), -100.0, None)
    l1p = jnp.clip(jnp.log(1.0 - scs), -100.0, None)
    d_ref[...] = -(tcs * lp + (1.0 - tcs) * l1p)


@jax.jit
def _stream(t_cls, s_cls, t_box, s_box, t_cent, s_cent):
    t2 = t_cls.reshape(R128, 128 * C_CLS)
    s2 = s_cls.reshape(R128, 128 * C_CLS)
    tb = t_box.reshape(R128, 128 * C_BOX)
    sb = s_box.reshape(R128, 128 * C_BOX)
    tc = t_cent.reshape(R128, 128)
    sc = s_cent.reshape(R128, 128)
    return pl.pallas_call(
        _stream_kernel,
        grid=(NB,),
        in_specs=[
            pl.BlockSpec((BR, 128 * C_CLS), lambda i: (i, 0)),
            pl.BlockSpec((BR, 128 * C_CLS), lambda i: (i, 0)),
            pl.BlockSpec((BR, 128 * C_BOX), lambda i: (i, 0)),
            pl.BlockSpec((BR, 128 * C_BOX), lambda i: (i, 0)),
            pl.BlockSpec((BR, 128), lambda i: (i, 0)),
            pl.BlockSpec((BR, 128), lambda i: (i, 0)),
        ],
        out_specs=[
            pl.BlockSpec((BR, 128), lambda i: (i, 0)),
            pl.BlockSpec((BR, 128), lambda i: (i, 0)),
            pl.BlockSpec((BR, 128), lambda i: (i, 0)),
            pl.BlockSpec((BR, 128), lambda i: (i, 0)),
            pl.BlockSpec((BR, 128), lambda i: (i, 0)),
            pl.BlockSpec(memory_space=pltpu.SMEM),
        ],
        out_shape=[
            jax.ShapeDtypeStruct((R128, 128), jnp.float32),
            jax.ShapeDtypeStruct((R128, 128), jnp.int32),
            jax.ShapeDtypeStruct((R128, 128), jnp.float32),
            jax.ShapeDtypeStruct((R128, 128), jnp.float32),
            jax.ShapeDtypeStruct((R128, 128), jnp.float32),
            jax.ShapeDtypeStruct((1, 1), jnp.float32),
        ],
        scratch_shapes=[
            pltpu.VMEM((128 * C_CLS, 128), jnp.float32),
            pltpu.VMEM((128 * C_BOX, 128), jnp.float32),
        ],
    )(t2, s2, tb, sb, tc, sc)


def _sc_body(maxv_hbm, maxbits_hbm, diff_hbm, c_hbm, d_hbm,
             thr_out, sgt_out, cgt_out, sdiff_out, sc_out, sd_out,
             vals, red, vh, vbh, dh, ch, ddh, tmp, ftmp, sem):
    cid = lax.axis_index("c")
    sid = lax.axis_index("s")
    zeros16 = jnp.zeros((16,), jnp.float32)
    img_l = sid // 2
    half = sid % 2
    img = cid * IMG_PER_CORE + img_l

    pltpu.sync_copy(maxbits_hbm.at[pl.ds(img * L, L)], vals)
    k_v = jnp.full((16,), K, jnp.int32)
    zi = jnp.zeros((16,), jnp.int32)
    oi = jnp.ones((16,), jnp.int32)
    lane = lax.iota(jnp.int32, 16)
    red[pl.ds(16, 16)] = zi

    def it(j, lohi):
        lo, hi = lohi
        mid = lo + lax.shift_right_logical(hi - lo, 1)

        def cnt(i, a):
            return a + jnp.where(vals[pl.ds(i * 16, 16)] >= mid, oi, zi)

        c = lax.fori_loop(0, NV_FULL, cnt, zi)
        s = c
        for k in (8, 4, 2, 1):
            red[pl.ds(0, 16)] = s
            s = s + red[pl.ds(k, 16)]
        acc = jnp.where(lane == 0, s, zi)
        for k in (1, 2, 4, 8):
            red[pl.ds(32, 16)] = zi
            red[pl.ds(48, 16)] = zi
            red[pl.ds(32 + k, 16)] = acc
            acc = acc + red[pl.ds(32, 16)]
        ge = acc >= k_v
        return (jnp.where(ge, mid, lo), jnp.where(ge, hi, mid))

    thr, _ = lax.fori_loop(
        0, 30, it,
        (jnp.zeros((16,), jnp.int32),
         jnp.full((16,), 0x40000000, jnp.int32)))

    @pl.when(half == 0)
    def _emit_thr():
        tmp[...] = thr
        pltpu.sync_copy(tmp, thr_out.at[cid, sid])

    base = img * L + half * HALF
    pltpu.sync_copy(maxv_hbm.at[pl.ds(base, HALF)], vh)
    pltpu.sync_copy(maxbits_hbm.at[pl.ds(base, HALF)], vbh)
    pltpu.sync_copy(diff_hbm.at[pl.ds(base, HALF)], dh)
    pltpu.sync_copy(c_hbm.at[pl.ds(base, HALF)], ch)
    pltpu.sync_copy(d_hbm.at[pl.ds(base, HALF)], ddh)
    onesf = jnp.ones((16,), jnp.float32)

    def acc(i, carry):
        s_gt, c_gt, s_diff, s_c, s_d = carry
        bits = vbh[pl.ds(i * 16, 16)]
        gt = bits > thr
        ge = bits >= thr
        v = vh[pl.ds(i * 16, 16)]
        s_gt = s_gt + jnp.where(gt, v, zeros16)
        c_gt = c_gt + jnp.where(gt, onesf, zeros16)
        s_diff = s_diff + jnp.where(ge, dh[pl.ds(i * 16, 16)], zeros16)
        s_c = s_c + jnp.where(ge, ch[pl.ds(i * 16, 16)], zeros16)
        s_d = s_d + jnp.where(ge, ddh[pl.ds(i * 16, 16)], zeros16)
        return (s_gt, c_gt, s_diff, s_c, s_d)

    s_gt, c_gt, s_diff, s_c, s_d = lax.fori_loop(
        0, NV_HALF, acc, (zeros16, zeros16, zeros16, zeros16, zeros16))

    ftmp[...] = s_gt
    pltpu.sync_copy(ftmp, sgt_out.at[cid, sid])
    ftmp[...] = c_gt
    pltpu.sync_copy(ftmp, cgt_out.at[cid, sid])
    ftmp[...] = s_diff
    pltpu.sync_copy(ftmp, sdiff_out.at[cid, sid])
    ftmp[...] = s_c
    pltpu.sync_copy(ftmp, sc_out.at[cid, sid])
    ftmp[...] = s_d
    pltpu.sync_copy(ftmp, sd_out.at[cid, sid])


@jax.jit
def _run(t_cls, t_box, t_cent, s_cls, s_box, s_cent):
    maxv, maxbits, diff, cc, dd, neg_tot = _stream(t_cls, s_cls, t_box,
                                                   s_box, t_cent, s_cent)
    f32 = jnp.float32
    i32 = jnp.int32
    outs = ([jax.ShapeDtypeStruct((SC_CORES, SC_SUB, 16), i32)]
            + [jax.ShapeDtypeStruct((SC_CORES, SC_SUB, 16), f32)] * 5)
    sc_fn = pl.kernel(
        _sc_body,
        out_type=outs,
        mesh=plsc.VectorSubcoreMesh(core_axis_name="c",
                                    subcore_axis_name="s"),
        scratch_types=[
            pltpu.VMEM((L,), i32),
            pltpu.VMEM((64,), i32),
            pltpu.VMEM((HALF,), f32),
            pltpu.VMEM((HALF,), i32),
            pltpu.VMEM((HALF,), f32),
            pltpu.VMEM((HALF,), f32),
            pltpu.VMEM((HALF,), f32),
            pltpu.VMEM((16,), i32),
            pltpu.VMEM((16,), f32),
            pltpu.SemaphoreType.DMA,
        ],
    )
    thr, s_gt, c_gt, s_diff, s_c, s_d = sc_fn(
        maxv.reshape(-1), maxbits.reshape(-1), diff.reshape(-1),
        cc.reshape(-1), dd.reshape(-1))

    thr_img = lax.bitcast_convert_type(
        thr[:, 0::2, 0].reshape(-1), f32)
    sgt_img = jnp.sum(s_gt.reshape(SC_CORES, IMG_PER_CORE, 2, 16),
                      axis=(2, 3))
    cgt_img = jnp.sum(c_gt.reshape(SC_CORES, IMG_PER_CORE, 2, 16),
                      axis=(2, 3))
    sum_top = jnp.sum(sgt_img.reshape(-1)
                      + (K - cgt_img.reshape(-1)) * thr_img)
    fg_num = 1e-06 + sum_top
    loss_cls = (neg_tot[0, 0] + jnp.sum(s_diff)) / fg_num
    loss_bbox = jnp.sum(s_c) / (N_POS * C_BOX)
    loss_cent = jnp.sum(s_d) / N_POS
    return loss_cls, loss_bbox, loss_cent


def kernel(t_cls_scores, t_bbox_preds, t_centernesses, s_cls_scores,
           s_bbox_preds, s_centernesses, num_per_img):
    return _run(t_cls_scores, t_bbox_preds, t_centernesses,
                s_cls_scores, s_bbox_preds, s_centernesses)


# speedup vs baseline: 1.1341x; 1.1341x over previous
import functools

import jax
import jax.numpy as jnp
from jax import lax
from jax.experimental import pallas as pl
from jax.experimental.pallas import tpu as pltpu
from jax.experimental.pallas import tpu_sc as plsc

N_IMG = 16
L = 16384
N = N_IMG * L
K = max(1, int(L * 0.02))
N_POS = N_IMG * K
C_CLS = 17
C_BOX = 5
R128 = N // 128
BR = 64
NB = R128 // BR

SC_CORES = 2
SC_SUB = 16
IMG_PER_CORE = N_IMG // SC_CORES
HALF = L // 2
NV_FULL = L // 16
NV_HALF = HALF // 16
NBINS = 1024


def _sigmoid(x):
    return jax.nn.sigmoid(x)


def _stream_kernel(t_cls_ref, s_cls_ref, t_box_ref, s_box_ref,
                   t_cent_ref, s_cent_ref,
                   max_ref, maxbits_ref, diff_ref, c_ref, d_ref, neg_ref,
                   w17_ref, w5_ref):
    i = pl.program_id(0)

    @pl.when(i == 0)
    def _init():
        r17 = lax.broadcasted_iota(jnp.int32, (128 * C_CLS, 128), 0)
        c17 = lax.broadcasted_iota(jnp.int32, (128 * C_CLS, 128), 1)
        w17_ref[...] = ((r17 >= C_CLS * c17) & (r17 < C_CLS * c17 + C_CLS)
                        ).astype(jnp.float32)
        r5 = lax.broadcasted_iota(jnp.int32, (128 * C_BOX, 128), 0)
        c5 = lax.broadcasted_iota(jnp.int32, (128 * C_BOX, 128), 1)
        w5_ref[...] = ((r5 >= C_BOX * c5) & (r5 < C_BOX * c5 + C_BOX)
                       ).astype(jnp.float32)
        neg_ref[0, 0] = 0.0

    t = t_cls_ref[...]
    s = s_cls_ref[...]
    s_sig = _sigmoid(s)
    t_sig = _sigmoid(t)
    ls = jnp.clip(jnp.log(s_sig), -100.0, None)
    l1s = jnp.clip(jnp.log(1.0 - s_sig), -100.0, None)
    neg = -l1s * (s_sig * s_sig)
    dts = t_sig - s_sig
    pos = -(t_sig * ls + (1.0 - t_sig) * l1s) * (dts * dts)
    neg_ref[0, 0] += jnp.sum(neg)
    diff_ref[...] = jnp.dot(pos - neg, w17_ref[...],
                            preferred_element_type=jnp.float32,
                            precision=lax.Precision.HIGHEST)

    lane = lax.broadcasted_iota(jnp.int32, (BR, 128 * C_CLS), 1)
    posn = lane % C_CLS
    m = t
    for d in (1, 2, 4, 8, 16):
        shifted = pltpu.roll(m, 128 * C_CLS - d, 1)
        m = jnp.maximum(m, jnp.where(posn <= C_CLS - 1 - d, shifted,
                                     -jnp.inf))
    head = jnp.where(posn == 0, m, 0.0)
    maxv = _sigmoid(jnp.dot(head, w17_ref[...],
                            preferred_element_type=jnp.float32,
                            precision=lax.Precision.HIGHEST))
    max_ref[...] = maxv
    maxbits_ref[...] = lax.bitcast_convert_type(maxv, jnp.int32)

    dbox = jnp.abs(s_box_ref[...] - t_box_ref[...])
    sl1 = jnp.where(dbox < 1.0, 0.5 * dbox * dbox, dbox - 0.5)
    tcs = _sigmoid(t_cent_ref[...])
    c_ref[...] = jnp.dot(sl1, w5_ref[...],
                         preferred_element_type=jnp.float32,
                            precision=lax.Precision.HIGHEST) * tcs

    scs = _sigmoid(s_cent_ref[...])
    lp = jnp.clip(jnp.log(scs), -100.0, None)
    l1p = jnp.clip(jnp.log(1.0 - scs), -100.0, None)
    d_ref[...] = -(tcs * lp + (1.0 - tcs) * l1p)


@jax.jit
def _stream(t_cls, s_cls, t_box, s_box, t_cent, s_cent):
    t2 = t_cls.reshape(R128, 128 * C_CLS)
    s2 = s_cls.reshape(R128, 128 * C_CLS)
    tb = t_box.reshape(R128, 128 * C_BOX)
    sb = s_box.reshape(R128, 128 * C_BOX)
    tc = t_cent.reshape(R128, 128)
    sc = s_cent.reshape(R128, 128)
    return pl.pallas_call(
        _stream_kernel,
        grid=(NB,),
        in_specs=[
            pl.BlockSpec((BR, 128 * C_CLS), lambda i: (i, 0)),
            pl.BlockSpec((BR, 128 * C_CLS), lambda i: (i, 0)),
            pl.BlockSpec((BR, 128 * C_BOX), lambda i: (i, 0)),
            pl.BlockSpec((BR, 128 * C_BOX), lambda i: (i, 0)),
            pl.BlockSpec((BR, 128), lambda i: (i, 0)),
            pl.BlockSpec((BR, 128), lambda i: (i, 0)),
        ],
        out_specs=[
            pl.BlockSpec((BR, 128), lambda i: (i, 0)),
            pl.BlockSpec((BR, 128), lambda i: (i, 0)),
            pl.BlockSpec((BR, 128), lambda i: (i, 0)),
            pl.BlockSpec((BR, 128), lambda i: (i, 0)),
            pl.BlockSpec((BR, 128), lambda i: (i, 0)),
            pl.BlockSpec(memory_space=pltpu.SMEM),
        ],
        out_shape=[
            jax.ShapeDtypeStruct((R128, 128), jnp.float32),
            jax.ShapeDtypeStruct((R128, 128), jnp.int32),
            jax.ShapeDtypeStruct((R128, 128), jnp.float32),
            jax.ShapeDtypeStruct((R128, 128), jnp.float32),
            jax.ShapeDtypeStruct((R128, 128), jnp.float32),
            jax.ShapeDtypeStruct((1, 1), jnp.float32),
        ],
        scratch_shapes=[
            pltpu.VMEM((128 * C_CLS, 128), jnp.float32),
            pltpu.VMEM((128 * C_BOX, 128), jnp.float32),
        ],
    )(t2, s2, tb, sb, tc, sc)


def _sc_body(maxv_hbm, maxbits_hbm, diff_hbm, c_hbm, d_hbm,
             thr_out, sgt_out, cgt_out, sdiff_out, sc_out, sd_out,
             vals, red, vh, vbh, dh, ch, ddh, tmp, ftmp, sem):
    cid = lax.axis_index("c")
    sid = lax.axis_index("s")
    zeros16 = jnp.zeros((16,), jnp.float32)
    img_l = sid // 2
    half = sid % 2
    img = cid * IMG_PER_CORE + img_l

    pltpu.sync_copy(maxbits_hbm.at[pl.ds(img * L, L)], vals)
    k_v = jnp.full((16,), K, jnp.int32)
    zi = jnp.zeros((16,), jnp.int32)
    oi = jnp.ones((16,), jnp.int32)
    lane = lax.iota(jnp.int32, 16)
    red[pl.ds(16, 16)] = zi

    def it(j, lohi):
        lo, hi = lohi
        mid = lo + lax.shift_right_logical(hi - lo, 1)

        def cnt(i, a):
            return a + jnp.where(vals[pl.ds(i * 16, 16)] >= mid, oi, zi)

        c = lax.fori_loop(0, NV_FULL, cnt, zi, unroll=8)
        s = c
        for k in (8, 4, 2, 1):
            red[pl.ds(0, 16)] = s
            s = s + red[pl.ds(k, 16)]
        acc = jnp.where(lane == 0, s, zi)
        for k in (1, 2, 4, 8):
            red[pl.ds(32, 16)] = zi
            red[pl.ds(48, 16)] = zi
            red[pl.ds(32 + k, 16)] = acc
            acc = acc + red[pl.ds(32, 16)]
        ge = acc >= k_v
        return (jnp.where(ge, mid, lo), jnp.where(ge, hi, mid))

    thr, _ = lax.fori_loop(
        0, 30, it,
        (jnp.zeros((16,), jnp.int32),
         jnp.full((16,), 0x40000000, jnp.int32)))

    @pl.when(half == 0)
    def _emit_thr():
        tmp[...] = thr
        pltpu.sync_copy(tmp, thr_out.at[cid, sid])

    base = img * L + half * HALF
    pltpu.sync_copy(maxv_hbm.at[pl.ds(base, HALF)], vh)
    pltpu.sync_copy(maxbits_hbm.at[pl.ds(base, HALF)], vbh)
    pltpu.sync_copy(diff_hbm.at[pl.ds(base, HALF)], dh)
    pltpu.sync_copy(c_hbm.at[pl.ds(base, HALF)], ch)
    pltpu.sync_copy(d_hbm.at[pl.ds(base, HALF)], ddh)
    onesf = jnp.ones((16,), jnp.float32)

    def acc(i, carry):
        s_gt, c_gt, s_diff, s_c, s_d = carry
        bits = vbh[pl.ds(i * 16, 16)]
        gt = bits > thr
        ge = bits >= thr
        v = vh[pl.ds(i * 16, 16)]
        s_gt = s_gt + jnp.where(gt, v, zeros16)
        c_gt = c_gt + jnp.where(gt, onesf, zeros16)
        s_diff = s_diff + jnp.where(ge, dh[pl.ds(i * 16, 16)], zeros16)
        s_c = s_c + jnp.where(ge, ch[pl.ds(i * 16, 16)], zeros16)
        s_d = s_d + jnp.where(ge, ddh[pl.ds(i * 16, 16)], zeros16)
        return (s_gt, c_gt, s_diff, s_c, s_d)

    s_gt, c_gt, s_diff, s_c, s_d = lax.fori_loop(
        0, NV_HALF, acc, (zeros16, zeros16, zeros16, zeros16, zeros16),
        unroll=4)

    ftmp[...] = s_gt
    pltpu.sync_copy(ftmp, sgt_out.at[cid, sid])
    ftmp[...] = c_gt
    pltpu.sync_copy(ftmp, cgt_out.at[cid, sid])
    ftmp[...] = s_diff
    pltpu.sync_copy(ftmp, sdiff_out.at[cid, sid])
    ftmp[...] = s_c
    pltpu.sync_copy(ftmp, sc_out.at[cid, sid])
    ftmp[...] = s_d
    pltpu.sync_copy(ftmp, sd_out.at[cid, sid])


@jax.jit
def _run(t_cls, t_box, t_cent, s_cls, s_box, s_cent):
    maxv, maxbits, diff, cc, dd, neg_tot = _stream(t_cls, s_cls, t_box,
                                                   s_box, t_cent, s_cent)
    f32 = jnp.float32
    i32 = jnp.int32
    outs = ([jax.ShapeDtypeStruct((SC_CORES, SC_SUB, 16), i32)]
            + [jax.ShapeDtypeStruct((SC_CORES, SC_SUB, 16), f32)] * 5)
    sc_fn = pl.kernel(
        _sc_body,
        out_type=outs,
        mesh=plsc.VectorSubcoreMesh(core_axis_name="c",
                                    subcore_axis_name="s"),
        scratch_types=[
            pltpu.VMEM((L,), i32),
            pltpu.VMEM((64,), i32),
            pltpu.VMEM((HALF,), f32),
            pltpu.VMEM((HALF,), i32),
            pltpu.VMEM((HALF,), f32),
            pltpu.VMEM((HALF,), f32),
            pltpu.VMEM((HALF,), f32),
            pltpu.VMEM((16,), i32),
            pltpu.VMEM((16,), f32),
            pltpu.SemaphoreType.DMA,
        ],
    )
    thr, s_gt, c_gt, s_diff, s_c, s_d = sc_fn(
        maxv.reshape(-1), maxbits.reshape(-1), diff.reshape(-1),
        cc.reshape(-1), dd.reshape(-1))

    thr_img = lax.bitcast_convert_type(
        thr[:, 0::2, 0].reshape(-1), f32)
    sgt_img = jnp.sum(s_gt.reshape(SC_CORES, IMG_PER_CORE, 2, 16),
                      axis=(2, 3))
    cgt_img = jnp.sum(c_gt.reshape(SC_CORES, IMG_PER_CORE, 2, 16),
                      axis=(2, 3))
    sum_top = jnp.sum(sgt_img.reshape(-1)
                      + (K - cgt_img.reshape(-1)) * thr_img)
    fg_num = 1e-06 + sum_top
    loss_cls = (neg_tot[0, 0] + jnp.sum(s_diff)) / fg_num
    loss_bbox = jnp.sum(s_c) / (N_POS * C_BOX)
    loss_cent = jnp.sum(s_d) / N_POS
    return loss_cls, loss_bbox, loss_cent


def kernel(t_cls_scores, t_bbox_preds, t_centernesses, s_cls_scores,
           s_bbox_preds, s_centernesses, num_per_img):
    return _run(t_cls_scores, t_bbox_preds, t_centernesses,
                s_cls_scores, s_bbox_preds, s_centernesses)


# speedup vs baseline: 1.1795x; 1.0400x over previous
import functools

import jax
import jax.numpy as jnp
from jax import lax
from jax.experimental import pallas as pl
from jax.experimental.pallas import tpu as pltpu
from jax.experimental.pallas import tpu_sc as plsc

N_IMG = 16
L = 16384
N = N_IMG * L
K = max(1, int(L * 0.02))
N_POS = N_IMG * K
C_CLS = 17
C_BOX = 5
R128 = N // 128
BR = 128
NB = R128 // BR

SC_CORES = 2
SC_SUB = 16
IMG_PER_CORE = N_IMG // SC_CORES
HALF = L // 2
NV_FULL = L // 16
NV_HALF = HALF // 16
NBINS = 1024


def _sigmoid(x):
    return jax.nn.sigmoid(x)


def _stream_kernel(t_cls_ref, s_cls_ref, t_box_ref, s_box_ref,
                   t_cent_ref, s_cent_ref,
                   max_ref, maxbits_ref, diff_ref, c_ref, d_ref, neg_ref,
                   w17_ref, w5_ref):
    i = pl.program_id(0)

    @pl.when(i == 0)
    def _init():
        r17 = lax.broadcasted_iota(jnp.int32, (128 * C_CLS, 128), 0)
        c17 = lax.broadcasted_iota(jnp.int32, (128 * C_CLS, 128), 1)
        w17_ref[...] = ((r17 >= C_CLS * c17) & (r17 < C_CLS * c17 + C_CLS)
                        ).astype(jnp.float32)
        r5 = lax.broadcasted_iota(jnp.int32, (128 * C_BOX, 128), 0)
        c5 = lax.broadcasted_iota(jnp.int32, (128 * C_BOX, 128), 1)
        w5_ref[...] = ((r5 >= C_BOX * c5) & (r5 < C_BOX * c5 + C_BOX)
                       ).astype(jnp.float32)
        neg_ref[0, 0] = 0.0

    t = t_cls_ref[...]
    s = s_cls_ref[...]
    s_sig = _sigmoid(s)
    t_sig = _sigmoid(t)
    ls = jnp.clip(jnp.log(s_sig), -100.0, None)
    l1s = jnp.clip(jnp.log(1.0 - s_sig), -100.0, None)
    neg = -l1s * (s_sig * s_sig)
    dts = t_sig - s_sig
    pos = -(t_sig * ls + (1.0 - t_sig) * l1s) * (dts * dts)
    neg_ref[0, 0] += jnp.sum(neg)
    diff_ref[...] = jnp.dot(pos - neg, w17_ref[...],
                            preferred_element_type=jnp.float32)

    lane = lax.broadcasted_iota(jnp.int32, (BR, 128 * C_CLS), 1)
    posn = lane % C_CLS
    m = t
    for d in (1, 2, 4, 8, 16):
        shifted = pltpu.roll(m, 128 * C_CLS - d, 1)
        m = jnp.maximum(m, jnp.where(posn <= C_CLS - 1 - d, shifted,
                                     -jnp.inf))
    head = jnp.where(posn == 0, m, 0.0)
    maxv = _sigmoid(jnp.dot(head, w17_ref[...],
                            preferred_element_type=jnp.float32,
                            precision=lax.Precision.HIGHEST))
    max_ref[...] = maxv
    maxbits_ref[...] = lax.bitcast_convert_type(maxv, jnp.int32)

    dbox = jnp.abs(s_box_ref[...] - t_box_ref[...])
    sl1 = jnp.where(dbox < 1.0, 0.5 * dbox * dbox, dbox - 0.5)
    tcs = _sigmoid(t_cent_ref[...])
    c_ref[...] = jnp.dot(sl1, w5_ref[...],
                         preferred_element_type=jnp.float32) * tcs

    scs = _sigmoid(s_cent_ref[...])
    lp = jnp.clip(jnp.log(scs), -100.0, None)
    l1p = jnp.clip(jnp.log(1.0 - scs), -100.0, None)
    d_ref[...] = -(tcs * lp + (1.0 - tcs) * l1p)


@jax.jit
def _stream(t_cls, s_cls, t_box, s_box, t_cent, s_cent):
    t2 = t_cls.reshape(R128, 128 * C_CLS)
    s2 = s_cls.reshape(R128, 128 * C_CLS)
    tb = t_box.reshape(R128, 128 * C_BOX)
    sb = s_box.reshape(R128, 128 * C_BOX)
    tc = t_cent.reshape(R128, 128)
    sc = s_cent.reshape(R128, 128)
    return pl.pallas_call(
        _stream_kernel,
        grid=(NB,),
        in_specs=[
            pl.BlockSpec((BR, 128 * C_CLS), lambda i: (i, 0)),
            pl.BlockSpec((BR, 128 * C_CLS), lambda i: (i, 0)),
            pl.BlockSpec((BR, 128 * C_BOX), lambda i: (i, 0)),
            pl.BlockSpec((BR, 128 * C_BOX), lambda i: (i, 0)),
            pl.BlockSpec((BR, 128), lambda i: (i, 0)),
            pl.BlockSpec((BR, 128), lambda i: (i, 0)),
        ],
        out_specs=[
            pl.BlockSpec((BR, 128), lambda i: (i, 0)),
            pl.BlockSpec((BR, 128), lambda i: (i, 0)),
            pl.BlockSpec((BR, 128), lambda i: (i, 0)),
            pl.BlockSpec((BR, 128), lambda i: (i, 0)),
            pl.BlockSpec((BR, 128), lambda i: (i, 0)),
            pl.BlockSpec(memory_space=pltpu.SMEM),
        ],
        out_shape=[
            jax.ShapeDtypeStruct((R128, 128), jnp.float32),
            jax.ShapeDtypeStruct((R128, 128), jnp.int32),
            jax.ShapeDtypeStruct((R128, 128), jnp.float32),
            jax.ShapeDtypeStruct((R128, 128), jnp.float32),
            jax.ShapeDtypeStruct((R128, 128), jnp.float32),
            jax.ShapeDtypeStruct((1, 1), jnp.float32),
        ],
        scratch_shapes=[
            pltpu.VMEM((128 * C_CLS, 128), jnp.float32),
            pltpu.VMEM((128 * C_BOX, 128), jnp.float32),
        ],
    )(t2, s2, tb, sb, tc, sc)


def _sc_body(maxv_hbm, maxbits_hbm, diff_hbm, c_hbm, d_hbm,
             thr_out, sgt_out, cgt_out, sdiff_out, sc_out, sd_out,
             vals, red, vh, vbh, dh, ch, ddh, tmp, ftmp, sem):
    cid = lax.axis_index("c")
    sid = lax.axis_index("s")
    zeros16 = jnp.zeros((16,), jnp.float32)
    img_l = sid // 2
    half = sid % 2
    img = cid * IMG_PER_CORE + img_l

    pltpu.sync_copy(maxbits_hbm.at[pl.ds(img * L, L)], vals)
    k_v = jnp.full((16,), K, jnp.int32)
    zi = jnp.zeros((16,), jnp.int32)
    oi = jnp.ones((16,), jnp.int32)
    lane = lax.iota(jnp.int32, 16)
    red[pl.ds(16, 16)] = zi

    def it(j, lohi):
        lo, hi = lohi
        mid = lo + lax.shift_right_logical(hi - lo, 1)

        def cnt(i, a):
            return a + jnp.where(vals[pl.ds(i * 16, 16)] >= mid, oi, zi)

        c = lax.fori_loop(0, NV_FULL, cnt, zi, unroll=8)
        s = c
        for k in (8, 4, 2, 1):
            red[pl.ds(0, 16)] = s
            s = s + red[pl.ds(k, 16)]
        acc = jnp.where(lane == 0, s, zi)
        for k in (1, 2, 4, 8):
            red[pl.ds(32, 16)] = zi
            red[pl.ds(48, 16)] = zi
            red[pl.ds(32 + k, 16)] = acc
            acc = acc + red[pl.ds(32, 16)]
        ge = acc >= k_v
        return (jnp.where(ge, mid, lo), jnp.where(ge, hi, mid))

    thr, _ = lax.fori_loop(
        0, 30, it,
        (jnp.zeros((16,), jnp.int32),
         jnp.full((16,), 0x40000000, jnp.int32)))

    @pl.when(half == 0)
    def _emit_thr():
        tmp[...] = thr
        pltpu.sync_copy(tmp, thr_out.at[cid, sid])

    base = img * L + half * HALF
    pltpu.sync_copy(maxv_hbm.at[pl.ds(base, HALF)], vh)
    pltpu.sync_copy(maxbits_hbm.at[pl.ds(base, HALF)], vbh)
    pltpu.sync_copy(diff_hbm.at[pl.ds(base, HALF)], dh)
    pltpu.sync_copy(c_hbm.at[pl.ds(base, HALF)], ch)
    pltpu.sync_copy(d_hbm.at[pl.ds(base, HALF)], ddh)
    onesf = jnp.ones((16,), jnp.float32)

    def acc(i, carry):
        s_gt, c_gt, s_diff, s_c, s_d = carry
        bits = vbh[pl.ds(i * 16, 16)]
        gt = bits > thr
        ge = bits >= thr
        v = vh[pl.ds(i * 16, 16)]
        s_gt = s_gt + jnp.where(gt, v, zeros16)
        c_gt = c_gt + jnp.where(gt, onesf, zeros16)
        s_diff = s_diff + jnp.where(ge, dh[pl.ds(i * 16, 16)], zeros16)
        s_c = s_c + jnp.where(ge, ch[pl.ds(i * 16, 16)], zeros16)
        s_d = s_d + jnp.where(ge, ddh[pl.ds(i * 16, 16)], zeros16)
        return (s_gt, c_gt, s_diff, s_c, s_d)

    s_gt, c_gt, s_diff, s_c, s_d = lax.fori_loop(
        0, NV_HALF, acc, (zeros16, zeros16, zeros16, zeros16, zeros16),
        unroll=4)

    ftmp[...] = s_gt
    pltpu.sync_copy(ftmp, sgt_out.at[cid, sid])
    ftmp[...] = c_gt
    pltpu.sync_copy(ftmp, cgt_out.at[cid, sid])
    ftmp[...] = s_diff
    pltpu.sync_copy(ftmp, sdiff_out.at[cid, sid])
    ftmp[...] = s_c
    pltpu.sync_copy(ftmp, sc_out.at[cid, sid])
    ftmp[...] = s_d
    pltpu.sync_copy(ftmp, sd_out.at[cid, sid])


@jax.jit
def _run(t_cls, t_box, t_cent, s_cls, s_box, s_cent):
    maxv, maxbits, diff, cc, dd, neg_tot = _stream(t_cls, s_cls, t_box,
                                                   s_box, t_cent, s_cent)
    f32 = jnp.float32
    i32 = jnp.int32
    outs = ([jax.ShapeDtypeStruct((SC_CORES, SC_SUB, 16), i32)]
            + [jax.ShapeDtypeStruct((SC_CORES, SC_SUB, 16), f32)] * 5)
    sc_fn = pl.kernel(
        _sc_body,
        out_type=outs,
        mesh=plsc.VectorSubcoreMesh(core_axis_name="c",
                                    subcore_axis_name="s"),
        scratch_types=[
            pltpu.VMEM((L,), i32),
            pltpu.VMEM((64,), i32),
            pltpu.VMEM((HALF,), f32),
            pltpu.VMEM((HALF,), i32),
            pltpu.VMEM((HALF,), f32),
            pltpu.VMEM((HALF,), f32),
            pltpu.VMEM((HALF,), f32),
            pltpu.VMEM((16,), i32),
            pltpu.VMEM((16,), f32),
            pltpu.SemaphoreType.DMA,
        ],
    )
    thr, s_gt, c_gt, s_diff, s_c, s_d = sc_fn(
        maxv.reshape(-1), maxbits.reshape(-1), diff.reshape(-1),
        cc.reshape(-1), dd.reshape(-1))

    thr_img = lax.bitcast_convert_type(
        thr[:, 0::2, 0].reshape(-1), f32)
    sgt_img = jnp.sum(s_gt.reshape(SC_CORES, IMG_PER_CORE, 2, 16),
                      axis=(2, 3))
    cgt_img = jnp.sum(c_gt.reshape(SC_CORES, IMG_PER_CORE, 2, 16),
                      axis=(2, 3))
    sum_top = jnp.sum(sgt_img.reshape(-1)
                      + (K - cgt_img.reshape(-1)) * thr_img)
    fg_num = 1e-06 + sum_top
    loss_cls = (neg_tot[0, 0] + jnp.sum(s_diff)) / fg_num
    loss_bbox = jnp.sum(s_c) / (N_POS * C_BOX)
    loss_cent = jnp.sum(s_d) / N_POS
    return loss_cls, loss_bbox, loss_cent


def kernel(t_cls_scores, t_bbox_preds, t_centernesses, s_cls_scores,
           s_bbox_preds, s_centernesses, num_per_img):
    return _run(t_cls_scores, t_bbox_preds, t_centernesses,
                s_cls_scores, s_bbox_preds, s_centernesses)


# speedup vs baseline: 6.0665x; 5.1435x over previous
import jax
import jax.numpy as jnp
from jax import lax
from jax.experimental import pallas as pl
from jax.experimental.pallas import tpu as pltpu
from jax.experimental.pallas import tpu_sc as plsc

N_IMG = 16
L = 16384
N = N_IMG * L
K = max(1, int(L * 0.02))
N_POS = N_IMG * K
C_CLS = 17
C_BOX = 5
BLKN = 16384
NB = N // BLKN

SC_CORES = 2
IMG_PER_CORE = N_IMG // SC_CORES
HALF = L // 2
NV_FULL = L // 16
NV_HALF = HALF // 16


def _sigmoid(x):
    return jax.nn.sigmoid(x)


def _stream_kernel(t_cls_ref, s_cls_ref, t_box_ref, s_box_ref,
                   t_cent_ref, s_cent_ref,
                   max_ref, maxbits_ref, diff_ref, c_ref, d_ref, neg_ref):
    i = pl.program_id(0)

    @pl.when(i == 0)
    def _init():
        neg_ref[0, 0] = 0.0

    t = t_cls_ref[...]
    s = s_cls_ref[...]
    s_sig = _sigmoid(s)
    t_sig = _sigmoid(t)
    ls = jnp.clip(jnp.log(s_sig), -100.0, None)
    l1s = jnp.clip(jnp.log(1.0 - s_sig), -100.0, None)
    neg = -l1s * (s_sig * s_sig)
    dts = t_sig - s_sig
    pos = -(t_sig * ls + (1.0 - t_sig) * l1s) * (dts * dts)
    neg_ref[0, 0] += jnp.sum(neg)
    diff_ref[...] = jnp.sum(pos - neg, axis=0)

    maxv = _sigmoid(jnp.max(t, axis=0))
    max_ref[...] = maxv
    maxbits_ref[...] = lax.bitcast_convert_type(maxv, jnp.int32)

    dbox = jnp.abs(s_box_ref[...] - t_box_ref[...])
    sl1 = jnp.where(dbox < 1.0, 0.5 * dbox * dbox, dbox - 0.5)
    tcs = _sigmoid(t_cent_ref[0, :])
    c_ref[...] = jnp.sum(sl1, axis=0) * tcs

    scs = _sigmoid(s_cent_ref[0, :])
    lp = jnp.clip(jnp.log(scs), -100.0, None)
    l1p = jnp.clip(jnp.log(1.0 - scs), -100.0, None)
    d_ref[...] = -(tcs * lp + (1.0 - tcs) * l1p)


@jax.jit
def _stream(t_cls, s_cls, t_box, s_box, t_cent, s_cent):
    f32 = jnp.float32
    return pl.pallas_call(
        _stream_kernel,
        grid=(NB,),
        in_specs=[
            pl.BlockSpec((C_CLS, BLKN), lambda i: (0, i)),
            pl.BlockSpec((C_CLS, BLKN), lambda i: (0, i)),
            pl.BlockSpec((C_BOX, BLKN), lambda i: (0, i)),
            pl.BlockSpec((C_BOX, BLKN), lambda i: (0, i)),
            pl.BlockSpec((1, BLKN), lambda i: (0, i)),
            pl.BlockSpec((1, BLKN), lambda i: (0, i)),
        ],
        out_specs=[
            pl.BlockSpec((BLKN,), lambda i: (i,)),
            pl.BlockSpec((BLKN,), lambda i: (i,)),
            pl.BlockSpec((BLKN,), lambda i: (i,)),
            pl.BlockSpec((BLKN,), lambda i: (i,)),
            pl.BlockSpec((BLKN,), lambda i: (i,)),
            pl.BlockSpec(memory_space=pltpu.SMEM),
        ],
        out_shape=[
            jax.ShapeDtypeStruct((N,), f32),
            jax.ShapeDtypeStruct((N,), jnp.int32),
            jax.ShapeDtypeStruct((N,), f32),
            jax.ShapeDtypeStruct((N,), f32),
            jax.ShapeDtypeStruct((N,), f32),
            jax.ShapeDtypeStruct((1, 1), f32),
        ],
    )(t_cls.T, s_cls.T, t_box.T, s_box.T,
      t_cent.reshape(1, N), s_cent.reshape(1, N))


def _sc_body(maxv_hbm, maxbits_hbm, diff_hbm, c_hbm, d_hbm,
             thr_out, sgt_out, cgt_out, sdiff_out, sc_out, sd_out,
             vals, red, vh, vbh, dh, ch, ddh, tmp, ftmp, sem):
    cid = lax.axis_index("c")
    sid = lax.axis_index("s")
    zeros16 = jnp.zeros((16,), jnp.float32)
    img_l = sid // 2
    half = sid % 2
    img = cid * IMG_PER_CORE + img_l

    pltpu.sync_copy(maxbits_hbm.at[pl.ds(img * L, L)], vals)
    k_v = jnp.full((16,), K, jnp.int32)
    zi = jnp.zeros((16,), jnp.int32)
    oi = jnp.ones((16,), jnp.int32)
    lane = lax.iota(jnp.int32, 16)
    red[pl.ds(16, 16)] = zi

    def it(j, lohi):
        lo, hi = lohi
        mid = lo + lax.shift_right_logical(hi - lo, 1)

        def cnt(i, a):
            return a + jnp.where(vals[pl.ds(i * 16, 16)] >= mid, oi, zi)

        c = lax.fori_loop(0, NV_FULL, cnt, zi, unroll=8)
        s = c
        for k in (8, 4, 2, 1):
            red[pl.ds(0, 16)] = s
            s = s + red[pl.ds(k, 16)]
        acc = jnp.where(lane == 0, s, zi)
        for k in (1, 2, 4, 8):
            red[pl.ds(32, 16)] = zi
            red[pl.ds(48, 16)] = zi
            red[pl.ds(32 + k, 16)] = acc
            acc = acc + red[pl.ds(32, 16)]
        ge = acc >= k_v
        return (jnp.where(ge, mid, lo), jnp.where(ge, hi, mid))

    thr, _ = lax.fori_loop(
        0, 30, it,
        (jnp.zeros((16,), jnp.int32),
         jnp.full((16,), 0x40000000, jnp.int32)))

    @pl.when(half == 0)
    def _emit_thr():
        tmp[...] = thr
        pltpu.sync_copy(tmp, thr_out.at[cid, sid])

    base = img * L + half * HALF
    pltpu.sync_copy(maxv_hbm.at[pl.ds(base, HALF)], vh)
    pltpu.sync_copy(maxbits_hbm.at[pl.ds(base, HALF)], vbh)
    pltpu.sync_copy(diff_hbm.at[pl.ds(base, HALF)], dh)
    pltpu.sync_copy(c_hbm.at[pl.ds(base, HALF)], ch)
    pltpu.sync_copy(d_hbm.at[pl.ds(base, HALF)], ddh)
    onesf = jnp.ones((16,), jnp.float32)

    def acc(i, carry):
        s_gt, c_gt, s_diff, s_c, s_d = carry
        bits = vbh[pl.ds(i * 16, 16)]
        gt = bits > thr
        ge = bits >= thr
        v = vh[pl.ds(i * 16, 16)]
        s_gt = s_gt + jnp.where(gt, v, zeros16)
        c_gt = c_gt + jnp.where(gt, onesf, zeros16)
        s_diff = s_diff + jnp.where(ge, dh[pl.ds(i * 16, 16)], zeros16)
        s_c = s_c + jnp.where(ge, ch[pl.ds(i * 16, 16)], zeros16)
        s_d = s_d + jnp.where(ge, ddh[pl.ds(i * 16, 16)], zeros16)
        return (s_gt, c_gt, s_diff, s_c, s_d)

    s_gt, c_gt, s_diff, s_c, s_d = lax.fori_loop(
        0, NV_HALF, acc, (zeros16, zeros16, zeros16, zeros16, zeros16),
        unroll=4)

    ftmp[...] = s_gt
    pltpu.sync_copy(ftmp, sgt_out.at[cid, sid])
    ftmp[...] = c_gt
    pltpu.sync_copy(ftmp, cgt_out.at[cid, sid])
    ftmp[...] = s_diff
    pltpu.sync_copy(ftmp, sdiff_out.at[cid, sid])
    ftmp[...] = s_c
    pltpu.sync_copy(ftmp, sc_out.at[cid, sid])
    ftmp[...] = s_d
    pltpu.sync_copy(ftmp, sd_out.at[cid, sid])


@jax.jit
def _run(t_cls, t_box, t_cent, s_cls, s_box, s_cent):
    maxv, maxbits, diff, cc, dd, neg_tot = _stream(t_cls, s_cls, t_box,
                                                   s_box, t_cent, s_cent)
    f32 = jnp.float32
    i32 = jnp.int32
    outs = ([jax.ShapeDtypeStruct((SC_CORES, 16, 16), i32)]
            + [jax.ShapeDtypeStruct((SC_CORES, 16, 16), f32)] * 5)
    sc_fn = pl.kernel(
        _sc_body,
        out_type=outs,
        mesh=plsc.VectorSubcoreMesh(core_axis_name="c",
                                    subcore_axis_name="s"),
        scratch_types=[
            pltpu.VMEM((L,), i32),
            pltpu.VMEM((64,), i32),
            pltpu.VMEM((HALF,), f32),
            pltpu.VMEM((HALF,), i32),
            pltpu.VMEM((HALF,), f32),
            pltpu.VMEM((HALF,), f32),
            pltpu.VMEM((HALF,), f32),
            pltpu.VMEM((16,), i32),
            pltpu.VMEM((16,), f32),
            pltpu.SemaphoreType.DMA,
        ],
    )
    thr, s_gt, c_gt, s_diff, s_c, s_d = sc_fn(maxv, maxbits, diff, cc, dd)

    thr_img = lax.bitcast_convert_type(thr[:, 0::2, 0].reshape(-1), f32)
    sgt_img = jnp.sum(s_gt.reshape(SC_CORES, IMG_PER_CORE, 2, 16),
                      axis=(2, 3))
    cgt_img = jnp.sum(c_gt.reshape(SC_CORES, IMG_PER_CORE, 2, 16),
                      axis=(2, 3))
    sum_top = jnp.sum(sgt_img.reshape(-1)
                      + (K - cgt_img.reshape(-1)) * thr_img)
    fg_num = 1e-06 + sum_top
    loss_cls = (neg_tot[0, 0] + jnp.sum(s_diff)) / fg_num
    loss_bbox = jnp.sum(s_c) / (N_POS * C_BOX)
    loss_cent = jnp.sum(s_d) / N_POS
    return loss_cls, loss_bbox, loss_cent


def kernel(t_cls_scores, t_bbox_preds, t_centernesses, s_cls_scores,
           s_bbox_preds, s_centernesses, num_per_img):
    return _run(t_cls_scores, t_bbox_preds, t_centernesses,
                s_cls_scores, s_bbox_preds, s_centernesses)


# speedup vs baseline: 6.1480x; 1.0134x over previous
import jax
import jax.numpy as jnp
from jax import lax
from jax.experimental import pallas as pl
from jax.experimental.pallas import tpu as pltpu
from jax.experimental.pallas import tpu_sc as plsc

N_IMG = 16
L = 16384
N = N_IMG * L
K = max(1, int(L * 0.02))
N_POS = N_IMG * K
C_CLS = 17
C_BOX = 5
BLKN = 8192
NB = N // BLKN

SC_CORES = 2
IMG_PER_CORE = N_IMG // SC_CORES
HALF = L // 2
NV_FULL = L // 16
NV_HALF = HALF // 16


def _sigmoid(x):
    return jax.nn.sigmoid(x)


def _stream_kernel(t_cls_ref, s_cls_ref, t_box_ref, s_box_ref,
                   t_cent_ref, s_cent_ref,
                   max_ref, maxbits_ref, diff_ref, c_ref, d_ref, neg_ref):
    i = pl.program_id(0)

    @pl.when(i == 0)
    def _init():
        neg_ref[0, 0] = 0.0

    t = t_cls_ref[...]
    s = s_cls_ref[...]
    s_sig = _sigmoid(s)
    t_sig = _sigmoid(t)
    ls = jnp.clip(jnp.log(s_sig), -100.0, None)
    l1s = jnp.clip(jnp.log(1.0 - s_sig), -100.0, None)
    neg = -l1s * (s_sig * s_sig)
    dts = t_sig - s_sig
    pos = -(t_sig * ls + (1.0 - t_sig) * l1s) * (dts * dts)
    neg_ref[0, 0] += jnp.sum(neg)
    diff_ref[...] = jnp.sum(pos - neg, axis=0)

    maxv = _sigmoid(jnp.max(t, axis=0))
    max_ref[...] = maxv
    maxbits_ref[...] = lax.bitcast_convert_type(maxv, jnp.int32)

    dbox = jnp.abs(s_box_ref[...] - t_box_ref[...])
    sl1 = jnp.where(dbox < 1.0, 0.5 * dbox * dbox, dbox - 0.5)
    tcs = _sigmoid(t_cent_ref[0, :])
    c_ref[...] = jnp.sum(sl1, axis=0) * tcs

    scs = _sigmoid(s_cent_ref[0, :])
    lp = jnp.clip(jnp.log(scs), -100.0, None)
    l1p = jnp.clip(jnp.log(1.0 - scs), -100.0, None)
    d_ref[...] = -(tcs * lp + (1.0 - tcs) * l1p)


@jax.jit
def _stream(t_cls, s_cls, t_box, s_box, t_cent, s_cent):
    f32 = jnp.float32
    return pl.pallas_call(
        _stream_kernel,
        grid=(NB,),
        in_specs=[
            pl.BlockSpec((C_CLS, BLKN), lambda i: (0, i)),
            pl.BlockSpec((C_CLS, BLKN), lambda i: (0, i)),
            pl.BlockSpec((C_BOX, BLKN), lambda i: (0, i)),
            pl.BlockSpec((C_BOX, BLKN), lambda i: (0, i)),
            pl.BlockSpec((1, BLKN), lambda i: (0, i)),
            pl.BlockSpec((1, BLKN), lambda i: (0, i)),
        ],
        out_specs=[
            pl.BlockSpec((BLKN,), lambda i: (i,)),
            pl.BlockSpec((BLKN,), lambda i: (i,)),
            pl.BlockSpec((BLKN,), lambda i: (i,)),
            pl.BlockSpec((BLKN,), lambda i: (i,)),
            pl.BlockSpec((BLKN,), lambda i: (i,)),
            pl.BlockSpec(memory_space=pltpu.SMEM),
        ],
        out_shape=[
            jax.ShapeDtypeStruct((N,), f32),
            jax.ShapeDtypeStruct((N,), jnp.int32),
            jax.ShapeDtypeStruct((N,), f32),
            jax.ShapeDtypeStruct((N,), f32),
            jax.ShapeDtypeStruct((N,), f32),
            jax.ShapeDtypeStruct((1, 1), f32),
        ],
    )(t_cls.T, s_cls.T, t_box.T, s_box.T,
      t_cent.reshape(1, N), s_cent.reshape(1, N))


def _sc_body(maxv_hbm, maxbits_hbm, diff_hbm, c_hbm, d_hbm,
             thr_out, sgt_out, cgt_out, sdiff_out, sc_out, sd_out,
             vals, red, vh, vbh, dh, ch, ddh, tmp, ftmp, sem):
    cid = lax.axis_index("c")
    sid = lax.axis_index("s")
    zeros16 = jnp.zeros((16,), jnp.float32)
    img_l = sid // 2
    half = sid % 2
    img = cid * IMG_PER_CORE + img_l

    pltpu.sync_copy(maxbits_hbm.at[pl.ds(img * L, L)], vals)
    k_v = jnp.full((16,), K, jnp.int32)
    zi = jnp.zeros((16,), jnp.int32)
    oi = jnp.ones((16,), jnp.int32)
    lane = lax.iota(jnp.int32, 16)
    red[pl.ds(16, 16)] = zi

    def it(j, lohi):
        lo, hi = lohi
        mid = lo + lax.shift_right_logical(hi - lo, 1)

        def cnt(i, a):
            return a + jnp.where(vals[pl.ds(i * 16, 16)] >= mid, oi, zi)

        c = lax.fori_loop(0, NV_FULL, cnt, zi, unroll=8)
        s = c
        for k in (8, 4, 2, 1):
            red[pl.ds(0, 16)] = s
            s = s + red[pl.ds(k, 16)]
        acc = jnp.where(lane == 0, s, zi)
        for k in (1, 2, 4, 8):
            red[pl.ds(32, 16)] = zi
            red[pl.ds(48, 16)] = zi
            red[pl.ds(32 + k, 16)] = acc
            acc = acc + red[pl.ds(32, 16)]
        ge = acc >= k_v
        return (jnp.where(ge, mid, lo), jnp.where(ge, hi, mid))

    thr, _ = lax.fori_loop(
        0, 30, it,
        (jnp.zeros((16,), jnp.int32),
         jnp.full((16,), 0x40000000, jnp.int32)))

    @pl.when(half == 0)
    def _emit_thr():
        tmp[...] = thr
        pltpu.sync_copy(tmp, thr_out.at[cid, sid])

    base = img * L + half * HALF
    pltpu.sync_copy(maxv_hbm.at[pl.ds(base, HALF)], vh)
    pltpu.sync_copy(maxbits_hbm.at[pl.ds(base, HALF)], vbh)
    pltpu.sync_copy(diff_hbm.at[pl.ds(base, HALF)], dh)
    pltpu.sync_copy(c_hbm.at[pl.ds(base, HALF)], ch)
    pltpu.sync_copy(d_hbm.at[pl.ds(base, HALF)], ddh)
    onesf = jnp.ones((16,), jnp.float32)

    def acc(i, carry):
        s_gt, c_gt, s_diff, s_c, s_d = carry
        bits = vbh[pl.ds(i * 16, 16)]
        gt = bits > thr
        ge = bits >= thr
        v = vh[pl.ds(i * 16, 16)]
        s_gt = s_gt + jnp.where(gt, v, zeros16)
        c_gt = c_gt + jnp.where(gt, onesf, zeros16)
        s_diff = s_diff + jnp.where(ge, dh[pl.ds(i * 16, 16)], zeros16)
        s_c = s_c + jnp.where(ge, ch[pl.ds(i * 16, 16)], zeros16)
        s_d = s_d + jnp.where(ge, ddh[pl.ds(i * 16, 16)], zeros16)
        return (s_gt, c_gt, s_diff, s_c, s_d)

    s_gt, c_gt, s_diff, s_c, s_d = lax.fori_loop(
        0, NV_HALF, acc, (zeros16, zeros16, zeros16, zeros16, zeros16),
        unroll=4)

    ftmp[...] = s_gt
    pltpu.sync_copy(ftmp, sgt_out.at[cid, sid])
    ftmp[...] = c_gt
    pltpu.sync_copy(ftmp, cgt_out.at[cid, sid])
    ftmp[...] = s_diff
    pltpu.sync_copy(ftmp, sdiff_out.at[cid, sid])
    ftmp[...] = s_c
    pltpu.sync_copy(ftmp, sc_out.at[cid, sid])
    ftmp[...] = s_d
    pltpu.sync_copy(ftmp, sd_out.at[cid, sid])


@jax.jit
def _run(t_cls, t_box, t_cent, s_cls, s_box, s_cent):
    maxv, maxbits, diff, cc, dd, neg_tot = _stream(t_cls, s_cls, t_box,
                                                   s_box, t_cent, s_cent)
    f32 = jnp.float32
    i32 = jnp.int32
    outs = ([jax.ShapeDtypeStruct((SC_CORES, 16, 16), i32)]
            + [jax.ShapeDtypeStruct((SC_CORES, 16, 16), f32)] * 5)
    sc_fn = pl.kernel(
        _sc_body,
        out_type=outs,
        mesh=plsc.VectorSubcoreMesh(core_axis_name="c",
                                    subcore_axis_name="s"),
        scratch_types=[
            pltpu.VMEM((L,), i32),
            pltpu.VMEM((64,), i32),
            pltpu.VMEM((HALF,), f32),
            pltpu.VMEM((HALF,), i32),
            pltpu.VMEM((HALF,), f32),
            pltpu.VMEM((HALF,), f32),
            pltpu.VMEM((HALF,), f32),
            pltpu.VMEM((16,), i32),
            pltpu.VMEM((16,), f32),
            pltpu.SemaphoreType.DMA,
        ],
    )
    thr, s_gt, c_gt, s_diff, s_c, s_d = sc_fn(maxv, maxbits, diff, cc, dd)

    thr_img = lax.bitcast_convert_type(thr[:, 0::2, 0].reshape(-1), f32)
    sgt_img = jnp.sum(s_gt.reshape(SC_CORES, IMG_PER_CORE, 2, 16),
                      axis=(2, 3))
    cgt_img = jnp.sum(c_gt.reshape(SC_CORES, IMG_PER_CORE, 2, 16),
                      axis=(2, 3))
    sum_top = jnp.sum(sgt_img.reshape(-1)
                      + (K - cgt_img.reshape(-1)) * thr_img)
    fg_num = 1e-06 + sum_top
    loss_cls = (neg_tot[0, 0] + jnp.sum(s_diff)) / fg_num
    loss_bbox = jnp.sum(s_c) / (N_POS * C_BOX)
    loss_cent = jnp.sum(s_d) / N_POS
    return loss_cls, loss_bbox, loss_cent


def kernel(t_cls_scores, t_bbox_preds, t_centernesses, s_cls_scores,
           s_bbox_preds, s_centernesses, num_per_img):
    return _run(t_cls_scores, t_bbox_preds, t_centernesses,
                s_cls_scores, s_bbox_preds, s_centernesses)
